# baseline (speedup 1.0000x reference)
import functools

import jax
import jax.numpy as jnp
from jax import lax
from jax.experimental import pallas as pl
from jax.experimental.pallas import tpu as pltpu
from jax.experimental.pallas import tpu_sc as plsc

NF = 4
NBIN = 32
GRID_SZ = NBIN ** NF
N_ST = 2_000_000
NT = 32
RB = 16384
NTC_A = 62
NTC_B = 61
N_A = NTC_A * RB
N_B = NTC_B * RB
GP_A = N_A // 128 // NT
GP_B = N_B // 128 // NT
MACG = 16
CHUNK = GRID_SZ // 16


def _bucketize(states, e4, tbl, w, base_block, nblocks, row0):
    def bk(x_ref, e_ref, t_ref, w_ref, o_ref):
        xr = jax.lax.dot(x_ref[...], e_ref[...],
                         precision=jax.lax.Precision.HIGHEST)
        c = jnp.where(t_ref[...] < xr, 1.0, 0.0)
        idx = jnp.dot(c, w_ref[...],
                      precision=jax.lax.Precision.HIGHEST).astype(jnp.int32)
        rid = (row0 + pl.program_id(0) * RB
               + lax.broadcasted_iota(jnp.int32, (RB,), 0))
        o_ref[...] = jnp.where(rid < N_ST, idx, GRID_SZ + (rid & 127))

    return pl.pallas_call(
        bk,
        grid=(nblocks,),
        in_specs=[
            pl.BlockSpec((RB, NF), lambda i: (base_block + i, 0)),
            pl.BlockSpec((NF, 128), lambda i: (0, 0)),
            pl.BlockSpec((1, 128), lambda i: (0, 0)),
            pl.BlockSpec((128,), lambda i: (0,)),
        ],
        out_specs=pl.BlockSpec((RB,), lambda i: (i,)),
        out_shape=jax.ShapeDtypeStruct((nblocks * RB,), jnp.int32),
    )(states, e4, tbl, w)


def _sc_hist(idx1d, g_per):
    nmac = (g_per + MACG - 1) // MACG
    mesh = plsc.VectorSubcoreMesh(core_axis_name="c", subcore_axis_name="s")

    @functools.partial(
        pl.kernel,
        mesh=mesh,
        out_type=jax.ShapeDtypeStruct((2, GRID_SZ), jnp.int32),
        compiler_params=pltpu.CompilerParams(needs_layout_passes=False),
        scratch_types=[
            pltpu.VMEM_SHARED((GRID_SZ + 128,), jnp.int32),
            pltpu.VMEM((MACG * 128,), jnp.int32),
            pltpu.VMEM((128,), jnp.int32),
            pltpu.VMEM((4096,), jnp.int32),
            pltpu.SemaphoreType.DMA,
        ],
    )
    def k(idx_hbm, out_hbm, grid, ibuf, ones, zbuf, ssem):
        c = lax.axis_index("c")
        s = lax.axis_index("s")
        wid = c * 16 + s

        one16 = jnp.ones((16,), jnp.int32)
        zero16 = jnp.zeros((16,), jnp.int32)
        for r in range(8):
            ones[pl.ds(16 * r, 16)] = one16

        def zb(i, carry):
            zbuf[pl.ds(16 * i, 16)] = zero16
            return carry

        lax.fori_loop(0, 256, zb, 0)

        def zc(i, carry):
            pltpu.sync_copy(zbuf, grid.at[pl.ds(s * CHUNK + i * 4096, 4096)])
            return carry

        lax.fori_loop(0, 16, zc, 0)
        plsc.subcore_barrier()

        gbase = g_per * wid

        def macro(m, carry):
            goff = jnp.minimum(MACG * m, g_per - MACG)
            gcnt = jnp.minimum(MACG, g_per - MACG * m)
            skip = MACG * m - goff
            pltpu.sync_copy(
                idx_hbm.at[pl.ds((gbase + goff) * 128, MACG * 128)], ibuf)

            def fire(j, carry2):
                pltpu.async_copy(
                    ones, grid.at[ibuf.at[pl.ds((skip + j) * 128, 128)]],
                    ssem, add=True)
                return carry2

            lax.fori_loop(0, gcnt, fire, 0)

            def drain(j, carry2):
                pltpu.make_async_copy(
                    ones, grid.at[ibuf.at[pl.ds((skip + j) * 128, 128)]],
                    ssem).wait()
                return carry2

            lax.fori_loop(0, gcnt, drain, 0)
            return carry

        lax.fori_loop(0, nmac, macro, 0)

        plsc.subcore_barrier()
        pltpu.sync_copy(grid.at[pl.ds(s * CHUNK, CHUNK)],
                        out_hbm.at[c, pl.ds(s * CHUNK, CHUNK)])

    return k(idx1d)


def _merge(counts2d, g0, g1, g2, g3):
    def mk(a_ref, b_ref, c_ref, d_ref, e_ref, o_ref):
        o_ref[...] = (a_ref[...] + b_ref[...] + c_ref[...]
                      + d_ref[...] + e_ref[...])

    return pl.pallas_call(
        mk,
        grid=(8,),
        in_specs=[pl.BlockSpec((128, 1024), lambda i: (i, 0))] * 5,
        out_specs=pl.BlockSpec((128, 1024), lambda i: (i, 0)),
        out_shape=jax.ShapeDtypeStruct((1024, 1024), jnp.int32),
    )(counts2d, g0, g1, g2, g3)


def kernel(states, b0, b1, b2, b3, counts):
    pad = jnp.full((1,), jnp.inf, dtype=jnp.float32)
    tbl = jnp.concatenate([b0, pad, b1, pad, b2, pad, b3, pad])
    lanes = jnp.arange(128, dtype=jnp.int32)
    seg = lanes // NBIN
    e4 = (seg[None, :] == jnp.arange(NF, dtype=jnp.int32)[:, None]
          ).astype(jnp.float32)
    w = jnp.float32(NBIN) ** (NF - 1 - seg).astype(jnp.float32)
    tbl2 = tbl.reshape(1, 128)
    idx_a = _bucketize(states, e4, tbl2, w, 0, NTC_A, 0)
    idx_b = _bucketize(states, e4, tbl2, w, NTC_A, NTC_B, N_A)
    pa = _sc_hist(idx_a, GP_A)
    pb = _sc_hist(idx_b, GP_B)
    merged = _merge(
        counts.reshape(1024, 1024),
        pa[0].reshape(1024, 1024),
        pa[1].reshape(1024, 1024),
        pb[0].reshape(1024, 1024),
        pb[1].reshape(1024, 1024),
    )
    return merged.reshape(NBIN, NBIN, NBIN, NBIN)


# speedup vs baseline: 1.0065x; 1.0065x over previous
import functools

import jax
import jax.numpy as jnp
from jax import lax
from jax.experimental import pallas as pl
from jax.experimental.pallas import tpu as pltpu
from jax.experimental.pallas import tpu_sc as plsc

NF = 4
NBIN = 32
GRID_SZ = NBIN ** NF
N_ST = 2_000_000
NT = 32
RB = 16384
NTC = 123
N_PAD = NTC * RB
G_TOT = N_PAD // 128
G_PER = G_TOT // NT
MACG = 16
NMAC = (G_PER + MACG - 1) // MACG
CHUNK = GRID_SZ // 16


def _bucketize(states, e4, tbl, w):
    def bk(x_ref, e_ref, t_ref, w_ref, o_ref):
        xr = jax.lax.dot(x_ref[...], e_ref[...],
                         precision=jax.lax.Precision.HIGHEST)
        c = jnp.where(t_ref[...] < xr, 1.0, 0.0)
        idx = jnp.dot(c, w_ref[...],
                      precision=jax.lax.Precision.HIGHEST).astype(jnp.int32)
        rid = pl.program_id(0) * RB + lax.broadcasted_iota(jnp.int32, (RB,), 0)
        o_ref[...] = jnp.where(rid < N_ST, idx, GRID_SZ + (rid & 127))

    return pl.pallas_call(
        bk,
        grid=(NTC,),
        in_specs=[
            pl.BlockSpec((RB, NF), lambda i: (i, 0)),
            pl.BlockSpec((NF, 128), lambda i: (0, 0)),
            pl.BlockSpec((1, 128), lambda i: (0, 0)),
            pl.BlockSpec((128,), lambda i: (0,)),
        ],
        out_specs=pl.BlockSpec((RB,), lambda i: (i,)),
        out_shape=jax.ShapeDtypeStruct((N_PAD,), jnp.int32),
    )(states, e4, tbl, w)


def _window(m):
    goff = jnp.minimum(MACG * m, G_PER - MACG)
    gcnt = jnp.minimum(MACG, G_PER - MACG * m)
    skip = MACG * m - goff
    return goff, gcnt, skip


def _sc_hist(idx1d):
    mesh = plsc.VectorSubcoreMesh(core_axis_name="c", subcore_axis_name="s")

    @functools.partial(
        pl.kernel,
        mesh=mesh,
        out_type=jax.ShapeDtypeStruct((2, GRID_SZ), jnp.int32),
        compiler_params=pltpu.CompilerParams(needs_layout_passes=False),
        scratch_types=[
            pltpu.VMEM_SHARED((GRID_SZ + 128,), jnp.int32),
            pltpu.VMEM((2, MACG * 128), jnp.int32),
            pltpu.VMEM((128,), jnp.int32),
            pltpu.VMEM((4096,), jnp.int32),
            pltpu.SemaphoreType.DMA,
            pltpu.SemaphoreType.DMA,
            pltpu.SemaphoreType.DMA,
        ],
    )
    def k(idx_hbm, out_hbm, grid, ibuf, ones, zbuf, ssem, zsem, psem):
        c = lax.axis_index("c")
        s = lax.axis_index("s")
        wid = c * 16 + s

        one16 = jnp.ones((16,), jnp.int32)
        zero16 = jnp.zeros((16,), jnp.int32)
        for r in range(8):
            ones[pl.ds(16 * r, 16)] = one16

        def zb(i, carry):
            zbuf[pl.ds(16 * i, 16)] = zero16
            return carry

        lax.fori_loop(0, 256, zb, 0)

        def zc(i, carry):
            pltpu.async_copy(zbuf, grid.at[pl.ds(s * CHUNK + i * 4096, 4096)],
                             zsem)
            return carry

        lax.fori_loop(0, 16, zc, 0)

        def zw(i, carry):
            pltpu.make_async_copy(
                zbuf, grid.at[pl.ds(s * CHUNK + i * 4096, 4096)], zsem).wait()
            return carry

        lax.fori_loop(0, 16, zw, 0)

        gbase = G_PER * wid

        def stage(m, slot):
            goff, _, _ = _window(m)
            pltpu.async_copy(
                idx_hbm.at[pl.ds((gbase + goff) * 128, MACG * 128)],
                ibuf.at[slot], psem)

        def stage_wait(m, slot):
            goff, _, _ = _window(m)
            pltpu.make_async_copy(
                idx_hbm.at[pl.ds((gbase + goff) * 128, MACG * 128)],
                ibuf.at[slot], psem).wait()

        stage(0, 0)

        def macro(m, carry):
            slot = m % 2
            stage_wait(m, slot)
            _, gcnt, skip = _window(m)

            def pre(cont):
                stage(m + 1, (m + 1) % 2)
                return cont

            _ = lax.cond(m + 1 < NMAC, lambda: pre(0), lambda: 0)

            def fire(j, carry2):
                pltpu.async_copy(
                    ones, grid.at[ibuf.at[slot, pl.ds((skip + j) * 128, 128)]],
                    ssem, add=True)
                return carry2

            lax.fori_loop(0, gcnt, fire, 0)

            def drain(j, carry2):
                pltpu.make_async_copy(
                    ones, grid.at[ibuf.at[slot, pl.ds((skip + j) * 128, 128)]],
                    ssem).wait()
                return carry2

            lax.fori_loop(0, gcnt, drain, 0)
            return carry

        lax.fori_loop(0, NMAC, macro, 0)

        plsc.subcore_barrier()
        pltpu.sync_copy(grid.at[pl.ds(s * CHUNK, CHUNK)],
                        out_hbm.at[c, pl.ds(s * CHUNK, CHUNK)])

    return k(idx1d)


def _merge(counts2d, g0, g1):
    def mk(a_ref, b_ref, c_ref, o_ref):
        o_ref[...] = a_ref[...] + b_ref[...] + c_ref[...]

    return pl.pallas_call(
        mk,
        grid=(8,),
        in_specs=[pl.BlockSpec((128, 1024), lambda i: (i, 0))] * 3,
        out_specs=pl.BlockSpec((128, 1024), lambda i: (i, 0)),
        out_shape=jax.ShapeDtypeStruct((1024, 1024), jnp.int32),
    )(counts2d, g0, g1)


def kernel(states, b0, b1, b2, b3, counts):
    pad = jnp.full((1,), jnp.inf, dtype=jnp.float32)
    tbl = jnp.concatenate([b0, pad, b1, pad, b2, pad, b3, pad])
    lanes = jnp.arange(128, dtype=jnp.int32)
    seg = lanes // NBIN
    e4 = (seg[None, :] == jnp.arange(NF, dtype=jnp.int32)[:, None]
          ).astype(jnp.float32)
    w = jnp.float32(NBIN) ** (NF - 1 - seg).astype(jnp.float32)
    idx1d = _bucketize(states, e4, tbl.reshape(1, 128), w)
    parts = _sc_hist(idx1d)
    merged = _merge(
        counts.reshape(1024, 1024),
        parts[0].reshape(1024, 1024),
        parts[1].reshape(1024, 1024),
    )
    return merged.reshape(NBIN, NBIN, NBIN, NBIN)
